# baseline (speedup 1.0000x reference)
import functools

import jax
import jax.numpy as jnp
from jax import lax
from jax.experimental import pallas as pl
from jax.experimental.pallas import tpu as pltpu
from jax.experimental.pallas import tpu_sc as plsc


def _make_sc_gather(B, d_cat):
    info = plsc.get_sparse_core_info()
    nw = info.num_cores * info.num_subcores
    b_per_w = B // nw
    mesh = plsc.VectorSubcoreMesh(core_axis_name="c", subcore_axis_name="s")

    f32 = jnp.float32
    out_type = [
        jax.ShapeDtypeStruct((B, d_cat), f32),
        jax.ShapeDtypeStruct((B, d_cat), f32),
    ]

    @functools.partial(
        pl.kernel,
        out_type=out_type,
        mesh=mesh,
        compiler_params=pltpu.CompilerParams(use_tc_tiling_on_sc=False),
        scratch_types=[
            pltpu.VMEM((b_per_w,), jnp.int32),
            pltpu.VMEM((b_per_w,), jnp.int32),
            pltpu.VMEM((b_per_w, d_cat), f32),
            pltpu.VMEM((b_per_w, d_cat), f32),
            pltpu.SemaphoreType.DMA,
            pltpu.SemaphoreType.DMA,
        ],
    )
    def gather_kernel(user_h, item_h, ucat_h, icat_h,
                      u_o, i_o, uidx, iidx, u_v, i_v, s1, s2):
        wid = lax.axis_index("s") * info.num_cores + lax.axis_index("c")
        base = wid * b_per_w
        pltpu.sync_copy(user_h.at[pl.ds(base, b_per_w)], uidx)
        pltpu.sync_copy(item_h.at[pl.ds(base, b_per_w)], iidx)
        c1 = pltpu.async_copy(ucat_h.at[uidx], u_v, s1)
        c2 = pltpu.async_copy(icat_h.at[iidx], i_v, s2)
        c1.wait()
        pltpu.sync_copy(u_v, u_o.at[pl.ds(base, b_per_w)])
        c2.wait()
        pltpu.sync_copy(i_v, i_o.at[pl.ds(base, b_per_w)])

    return gather_kernel


def _mlp_body(ug, ig, W1, b1, W2, b2, W3, b3, W4, b4, Wp, bp, out,
              *, d_mf, d_mlp):
    mfu = ug[:, 0:d_mf]
    mfi = ig[:, 0:d_mf]
    xu = ug[:, d_mf:d_mf + d_mlp]
    xi = ig[:, d_mf:d_mf + d_mlp]
    h = xu @ W1[0:d_mlp, :] + xi @ W1[d_mlp:2 * d_mlp, :] + b1[...]
    h = jnp.maximum(h, 0.0)
    h = jnp.maximum(h @ W2[...] + b2[...], 0.0)
    h = jnp.maximum(h @ W3[...] + b3[...], 0.0)
    h = jnp.maximum(h @ W4[...] + b4[...], 0.0)
    mf = mfu * mfi
    logit = mf @ Wp[0:d_mf, :] + h @ Wp[d_mf:, :] + bp[...]
    out[...] = 1.0 / (1.0 + jnp.exp(-logit))


def kernel(user, item, additional_features, mf_user_emb, mf_item_emb,
           mlp_user_emb, mlp_item_emb, W1, b1, W2, b2, W3, b3, W4, b4,
           Wp, bp):
    del additional_features
    B = user.shape[0]
    d_mf = mf_user_emb.shape[1]
    d_mlp = mlp_user_emb.shape[1]
    d_cat = d_mf + d_mlp

    ucat = jnp.concatenate([mf_user_emb, mlp_user_emb], axis=1)
    icat = jnp.concatenate([mf_item_emb, mlp_item_emb], axis=1)

    gather = _make_sc_gather(B, d_cat)
    ug, ig = gather(user, item, ucat, icat)

    blk = 2048
    full = lambda shape: pl.BlockSpec(shape, lambda i: (0, 0))
    body = functools.partial(_mlp_body, d_mf=d_mf, d_mlp=d_mlp)
    out = pl.pallas_call(
        body,
        grid=(B // blk,),
        in_specs=[
            pl.BlockSpec((blk, d_cat), lambda i: (i, 0)),
            pl.BlockSpec((blk, d_cat), lambda i: (i, 0)),
            full(W1.shape), full((1, b1.shape[0])),
            full(W2.shape), full((1, b2.shape[0])),
            full(W3.shape), full((1, b3.shape[0])),
            full(W4.shape), full((1, b4.shape[0])),
            full(Wp.shape), full((1, 1)),
        ],
        out_specs=pl.BlockSpec((blk, 1), lambda i: (i, 0)),
        out_shape=jax.ShapeDtypeStruct((B, 1), jnp.float32),
    )(ug, ig,
      W1, b1.reshape(1, -1), W2, b2.reshape(1, -1),
      W3, b3.reshape(1, -1), W4, b4.reshape(1, -1),
      Wp, bp.reshape(1, 1))
    return out.reshape(-1)


# speedup vs baseline: 3.3463x; 3.3463x over previous
import functools

import jax
import jax.numpy as jnp
from jax import lax
from jax.experimental import pallas as pl
from jax.experimental.pallas import tpu as pltpu
from jax.experimental.pallas import tpu_sc as plsc


def _make_sc_gather(B, d_mf, d_mlp, group):
    info = plsc.get_sparse_core_info()
    nw = info.num_cores * info.num_subcores
    b_per_w = B // nw
    half = b_per_w // 2
    mesh = plsc.VectorSubcoreMesh(core_axis_name="c", subcore_axis_name="s")

    f32 = jnp.float32
    out_type = [
        jax.ShapeDtypeStruct((B, 16), f32),
        jax.ShapeDtypeStruct((B, d_mlp), f32),
        jax.ShapeDtypeStruct((B, d_mlp), f32),
    ]

    @functools.partial(
        pl.kernel,
        out_type=out_type,
        mesh=mesh,
        compiler_params=pltpu.CompilerParams(use_tc_tiling_on_sc=True,
                                             needs_layout_passes=False),
        scratch_types=[
            pltpu.VMEM((b_per_w,), jnp.int32),
            pltpu.VMEM((b_per_w,), jnp.int32),
            pltpu.VMEM((d_mf, 128), f32),
            pltpu.VMEM((d_mf, 128), f32),
            pltpu.VMEM((d_mlp, 128), f32),
            pltpu.VMEM((d_mlp, 128), f32),
            pltpu.VMEM((half, 16), f32),
            pltpu.VMEM((half, d_mlp), f32),
            pltpu.VMEM((half, d_mlp), f32),
            pltpu.SemaphoreType.DMA,
            pltpu.SemaphoreType.DMA,
            pltpu.SemaphoreType.DMA,
            pltpu.SemaphoreType.DMA,
        ],
    )
    def gather_kernel(user_h, item_h, mfu_h, mfi_h, mlpu_h, mlpi_h,
                      mf_o, mlpu_o, mlpi_o,
                      uidx, iidx, mfu_s, mfi_s, mlpu_s, mlpi_s,
                      amf, amlpu, amlpi,
                      s1, s2, s3, s4):
        wid = lax.axis_index("s") * info.num_cores + lax.axis_index("c")
        base = wid * b_per_w
        pltpu.sync_copy(user_h.at[pl.ds(base, b_per_w)], uidx)
        pltpu.sync_copy(item_h.at[pl.ds(base, b_per_w)], iidx)

        rows16 = lax.iota(jnp.int32, 16)
        rows8 = rows16 & 7

        def half_pass(h, carry):
            h0 = h * half

            def body(jb, c2):
                g0 = h0 + jb * group
                a0 = jb * group
                uvec = uidx[pl.ds(g0, group)]
                ivec = iidx[pl.ds(g0, group)]
                ublk = (uvec >> 7) << 7
                iblk = (ivec >> 7) << 7
                ulane = uvec & 127
                ilane = ivec & 127
                for k in range(group):
                    j = a0 + k
                    ub = pl.multiple_of(ublk[k], 128)
                    ib = pl.multiple_of(iblk[k], 128)
                    c1 = pltpu.async_copy(
                        mfu_h.at[:, pl.ds(ub, 128)], mfu_s, s1)
                    cc2 = pltpu.async_copy(
                        mfi_h.at[:, pl.ds(ib, 128)], mfi_s, s2)
                    c3 = pltpu.async_copy(
                        mlpu_h.at[:, pl.ds(ub, 128)], mlpu_s, s3)
                    c4 = pltpu.async_copy(
                        mlpi_h.at[:, pl.ds(ib, 128)], mlpi_s, s4)
                    ul = jnp.full((16,), ulane[k], jnp.int32)
                    il = jnp.full((16,), ilane[k], jnp.int32)
                    c1.wait()
                    vmfu = plsc.load_gather(mfu_s, [rows8, ul])
                    cc2.wait()
                    vmfi = plsc.load_gather(mfi_s, [rows8, il])
                    amf[j, pl.ds(0, 16)] = jnp.where(rows16 < 8, vmfu, vmfi)
                    c3.wait()
                    amlpu[j, pl.ds(0, 16)] = plsc.load_gather(
                        mlpu_s, [rows16, ul])
                    amlpu[j, pl.ds(16, 16)] = plsc.load_gather(
                        mlpu_s, [rows16 + 16, ul])
                    c4.wait()
                    amlpi[j, pl.ds(0, 16)] = plsc.load_gather(
                        mlpi_s, [rows16, il])
                    amlpi[j, pl.ds(16, 16)] = plsc.load_gather(
                        mlpi_s, [rows16 + 16, il])
                return c2

            lax.fori_loop(0, half // group, body, 0)
            pltpu.sync_copy(amf, mf_o.at[pl.ds(base + h0, half)])
            pltpu.sync_copy(amlpu, mlpu_o.at[pl.ds(base + h0, half)])
            pltpu.sync_copy(amlpi, mlpi_o.at[pl.ds(base + h0, half)])
            return carry

        lax.fori_loop(0, 2, half_pass, 0)

    return gather_kernel


def _mlp_body(mf_ui, mlpu, mlpi, W1, b1, W2, b2, W3, b3, W4, b4, Wp, bp,
              out, *, d_mf, d_mlp):
    h = (mlpu[...] @ W1[0:d_mlp, :] + mlpi[...] @ W1[d_mlp:2 * d_mlp, :]
         + b1[...])
    h = jnp.maximum(h, 0.0)
    h = jnp.maximum(h @ W2[...] + b2[...], 0.0)
    h = jnp.maximum(h @ W3[...] + b3[...], 0.0)
    h = jnp.maximum(h @ W4[...] + b4[...], 0.0)
    mf = mf_ui[:, 0:d_mf] * mf_ui[:, d_mf:2 * d_mf]
    logit = mf @ Wp[0:d_mf, :] + h @ Wp[d_mf:, :] + bp[...]
    out[...] = 1.0 / (1.0 + jnp.exp(-logit))


def kernel(user, item, additional_features, mf_user_emb, mf_item_emb,
           mlp_user_emb, mlp_item_emb, W1, b1, W2, b2, W3, b3, W4, b4,
           Wp, bp):
    del additional_features
    B = user.shape[0]
    d_mf = mf_user_emb.shape[1]
    d_mlp = mlp_user_emb.shape[1]

    gather = _make_sc_gather(B, d_mf, d_mlp, group=16)
    mf_ui, mlpu, mlpi = gather(
        user, item, mf_user_emb.T, mf_item_emb.T,
        mlp_user_emb.T, mlp_item_emb.T)

    blk = 2048
    full = lambda shape: pl.BlockSpec(shape, lambda i: (0, 0))
    body = functools.partial(_mlp_body, d_mf=d_mf, d_mlp=d_mlp)
    out = pl.pallas_call(
        body,
        grid=(B // blk,),
        in_specs=[
            pl.BlockSpec((blk, 16), lambda i: (i, 0)),
            pl.BlockSpec((blk, d_mlp), lambda i: (i, 0)),
            pl.BlockSpec((blk, d_mlp), lambda i: (i, 0)),
            full(W1.shape), full((1, b1.shape[0])),
            full(W2.shape), full((1, b2.shape[0])),
            full(W3.shape), full((1, b3.shape[0])),
            full(W4.shape), full((1, b4.shape[0])),
            full(Wp.shape), full((1, 1)),
        ],
        out_specs=pl.BlockSpec((blk, 1), lambda i: (i, 0)),
        out_shape=jax.ShapeDtypeStruct((B, 1), jnp.float32),
    )(mf_ui, mlpu, mlpi,
      W1, b1.reshape(1, -1), W2, b2.reshape(1, -1),
      W3, b3.reshape(1, -1), W4, b4.reshape(1, -1),
      Wp, bp.reshape(1, 1))
    return out.reshape(-1)


# speedup vs baseline: 4.9796x; 1.4881x over previous
import functools

import jax
import jax.numpy as jnp
from jax import lax
from jax.experimental import pallas as pl
from jax.experimental.pallas import tpu as pltpu
from jax.experimental.pallas import tpu_sc as plsc


def _make_sc_gather(B, d_mf, d_mlp, group):
    info = plsc.get_sparse_core_info()
    nw = info.num_cores * info.num_subcores
    b_per_w = B // nw
    half = b_per_w // 2
    mesh = plsc.VectorSubcoreMesh(core_axis_name="c", subcore_axis_name="s")

    f32 = jnp.float32
    out_type = [
        jax.ShapeDtypeStruct((B, 16), f32),
        jax.ShapeDtypeStruct((B, d_mlp), f32),
        jax.ShapeDtypeStruct((B, d_mlp), f32),
    ]

    @functools.partial(
        pl.kernel,
        out_type=out_type,
        mesh=mesh,
        compiler_params=pltpu.CompilerParams(use_tc_tiling_on_sc=True,
                                             needs_layout_passes=False),
        scratch_types=[
            pltpu.VMEM((b_per_w,), jnp.int32),
            pltpu.VMEM((b_per_w,), jnp.int32),
            [pltpu.VMEM((d_mf, 128), f32)] * 2,
            [pltpu.VMEM((d_mf, 128), f32)] * 2,
            [pltpu.VMEM((d_mlp, 128), f32)] * 2,
            [pltpu.VMEM((d_mlp, 128), f32)] * 2,
            pltpu.VMEM((half, 16), f32),
            pltpu.VMEM((half, d_mlp), f32),
            pltpu.VMEM((half, d_mlp), f32),
            pltpu.SemaphoreType.DMA,
            pltpu.SemaphoreType.DMA,
            pltpu.SemaphoreType.DMA,
            pltpu.SemaphoreType.DMA,
        ],
    )
    def gather_kernel(user_h, item_h, mfu_h, mfi_h, mlpu_h, mlpi_h,
                      mf_o, mlpu_o, mlpi_o,
                      uidx, iidx, mfu_s, mfi_s, mlpu_s, mlpi_s,
                      amf, amlpu, amlpi,
                      s1, s2, s3, s4):
        wid = lax.axis_index("s") * info.num_cores + lax.axis_index("c")
        base = wid * b_per_w
        pltpu.sync_copy(user_h.at[pl.ds(base, b_per_w)], uidx)
        pltpu.sync_copy(item_h.at[pl.ds(base, b_per_w)], iidx)

        rows16 = lax.iota(jnp.int32, 16)
        rows8 = rows16 & 7

        def half_pass(h, carry):
            h0 = h * half

            def fire(ublk, iblk, k):
                ub = pl.multiple_of(ublk[k], 128)
                ib = pl.multiple_of(iblk[k], 128)
                q = k % 2
                return (
                    pltpu.async_copy(
                        mfu_h.at[:, pl.ds(ub, 128)], mfu_s[q], s1),
                    pltpu.async_copy(
                        mfi_h.at[:, pl.ds(ib, 128)], mfi_s[q], s2),
                    pltpu.async_copy(
                        mlpu_h.at[:, pl.ds(ub, 128)], mlpu_s[q], s3),
                    pltpu.async_copy(
                        mlpi_h.at[:, pl.ds(ib, 128)], mlpi_s[q], s4),
                )

            def body(jb, c2):
                g0 = h0 + jb * group
                a0 = jb * group
                uvec = uidx[pl.ds(g0, group)]
                ivec = iidx[pl.ds(g0, group)]
                ublk = (uvec >> 7) << 7
                iblk = (ivec >> 7) << 7
                ulane = uvec & 127
                ilane = ivec & 127
                cps = fire(ublk, iblk, 0)
                for k in range(group):
                    j = a0 + k
                    q = k % 2
                    c1, cc2, c3, c4 = cps
                    if k + 1 < group:
                        cps = fire(ublk, iblk, k + 1)
                    ul = jnp.full((16,), ulane[k], jnp.int32)
                    il = jnp.full((16,), ilane[k], jnp.int32)
                    c1.wait()
                    vmfu = plsc.load_gather(mfu_s[q], [rows8, ul])
                    cc2.wait()
                    vmfi = plsc.load_gather(mfi_s[q], [rows8, il])
                    amf[j, pl.ds(0, 16)] = jnp.where(rows16 < 8, vmfu, vmfi)
                    c3.wait()
                    amlpu[j, pl.ds(0, 16)] = plsc.load_gather(
                        mlpu_s[q], [rows16, ul])
                    amlpu[j, pl.ds(16, 16)] = plsc.load_gather(
                        mlpu_s[q], [rows16 + 16, ul])
                    c4.wait()
                    amlpi[j, pl.ds(0, 16)] = plsc.load_gather(
                        mlpi_s[q], [rows16, il])
                    amlpi[j, pl.ds(16, 16)] = plsc.load_gather(
                        mlpi_s[q], [rows16 + 16, il])
                return c2

            lax.fori_loop(0, half // group, body, 0)
            pltpu.sync_copy(amf, mf_o.at[pl.ds(base + h0, half)])
            pltpu.sync_copy(amlpu, mlpu_o.at[pl.ds(base + h0, half)])
            pltpu.sync_copy(amlpi, mlpi_o.at[pl.ds(base + h0, half)])
            return carry

        lax.fori_loop(0, 2, half_pass, 0)

    return gather_kernel


def _mlp_body(mf_ui, mlpu, mlpi, W1, b1, W2, b2, W3, b3, W4, b4, Wp, bp,
              out, *, d_mf, d_mlp):
    h = (mlpu[...] @ W1[0:d_mlp, :] + mlpi[...] @ W1[d_mlp:2 * d_mlp, :]
         + b1[...])
    h = jnp.maximum(h, 0.0)
    h = jnp.maximum(h @ W2[...] + b2[...], 0.0)
    h = jnp.maximum(h @ W3[...] + b3[...], 0.0)
    h = jnp.maximum(h @ W4[...] + b4[...], 0.0)
    mf = mf_ui[:, 0:d_mf] * mf_ui[:, d_mf:2 * d_mf]
    logit = mf @ Wp[0:d_mf, :] + h @ Wp[d_mf:, :] + bp[...]
    out[...] = 1.0 / (1.0 + jnp.exp(-logit))


def kernel(user, item, additional_features, mf_user_emb, mf_item_emb,
           mlp_user_emb, mlp_item_emb, W1, b1, W2, b2, W3, b3, W4, b4,
           Wp, bp):
    del additional_features
    B = user.shape[0]
    d_mf = mf_user_emb.shape[1]
    d_mlp = mlp_user_emb.shape[1]

    gather = _make_sc_gather(B, d_mf, d_mlp, group=16)
    mf_ui, mlpu, mlpi = gather(
        user, item, mf_user_emb.T, mf_item_emb.T,
        mlp_user_emb.T, mlp_item_emb.T)

    blk = 2048
    full = lambda shape: pl.BlockSpec(shape, lambda i: (0, 0))
    body = functools.partial(_mlp_body, d_mf=d_mf, d_mlp=d_mlp)
    out = pl.pallas_call(
        body,
        grid=(B // blk,),
        in_specs=[
            pl.BlockSpec((blk, 16), lambda i: (i, 0)),
            pl.BlockSpec((blk, d_mlp), lambda i: (i, 0)),
            pl.BlockSpec((blk, d_mlp), lambda i: (i, 0)),
            full(W1.shape), full((1, b1.shape[0])),
            full(W2.shape), full((1, b2.shape[0])),
            full(W3.shape), full((1, b3.shape[0])),
            full(W4.shape), full((1, b4.shape[0])),
            full(Wp.shape), full((1, 1)),
        ],
        out_specs=pl.BlockSpec((blk, 1), lambda i: (i, 0)),
        out_shape=jax.ShapeDtypeStruct((B, 1), jnp.float32),
    )(mf_ui, mlpu, mlpi,
      W1, b1.reshape(1, -1), W2, b2.reshape(1, -1),
      W3, b3.reshape(1, -1), W4, b4.reshape(1, -1),
      Wp, bp.reshape(1, 1))
    return out.reshape(-1)


# speedup vs baseline: 5.5490x; 1.1144x over previous
import functools

import jax
import jax.numpy as jnp
from jax import lax
from jax.experimental import pallas as pl
from jax.experimental.pallas import tpu as pltpu
from jax.experimental.pallas import tpu_sc as plsc


def _make_sc_gather(B, d_mf, d_mlp, group):
    info = plsc.get_sparse_core_info()
    nw = info.num_cores * info.num_subcores
    b_per_w = B // nw
    half = b_per_w // 2
    mesh = plsc.VectorSubcoreMesh(core_axis_name="c", subcore_axis_name="s")

    f32 = jnp.float32
    out_type = [
        jax.ShapeDtypeStruct((B, 16), f32),
        jax.ShapeDtypeStruct((B, d_mlp), f32),
        jax.ShapeDtypeStruct((B, d_mlp), f32),
    ]

    @functools.partial(
        pl.kernel,
        out_type=out_type,
        mesh=mesh,
        compiler_params=pltpu.CompilerParams(use_tc_tiling_on_sc=True,
                                             needs_layout_passes=False),
        scratch_types=[
            pltpu.VMEM((b_per_w,), jnp.int32),
            pltpu.VMEM((b_per_w,), jnp.int32),
            [pltpu.VMEM((d_mf, 128), f32)] * 3,
            [pltpu.VMEM((d_mf, 128), f32)] * 3,
            [pltpu.VMEM((d_mlp, 128), f32)] * 3,
            [pltpu.VMEM((d_mlp, 128), f32)] * 3,
            pltpu.VMEM((half, 16), f32),
            pltpu.VMEM((half, d_mlp), f32),
            pltpu.VMEM((half, d_mlp), f32),
            pltpu.SemaphoreType.DMA,
            pltpu.SemaphoreType.DMA,
            pltpu.SemaphoreType.DMA,
            pltpu.SemaphoreType.DMA,
        ],
    )
    def gather_kernel(user_h, item_h, mfu_h, mfi_h, mlpu_h, mlpi_h,
                      mf_o, mlpu_o, mlpi_o,
                      uidx, iidx, mfu_s, mfi_s, mlpu_s, mlpi_s,
                      amf, amlpu, amlpi,
                      s1, s2, s3, s4):
        wid = lax.axis_index("s") * info.num_cores + lax.axis_index("c")
        base = wid * b_per_w
        pltpu.sync_copy(user_h.at[pl.ds(base, b_per_w)], uidx)
        pltpu.sync_copy(item_h.at[pl.ds(base, b_per_w)], iidx)

        rows16 = lax.iota(jnp.int32, 16)
        rows8 = rows16 & 7

        def half_pass(h, carry):
            h0 = h * half

            def fire(ublk, iblk, k):
                ub = pl.multiple_of(ublk[k], 128)
                ib = pl.multiple_of(iblk[k], 128)
                q = k % 3
                return (
                    pltpu.async_copy(
                        mfu_h.at[:, pl.ds(ub, 128)], mfu_s[q], s1),
                    pltpu.async_copy(
                        mfi_h.at[:, pl.ds(ib, 128)], mfi_s[q], s2),
                    pltpu.async_copy(
                        mlpu_h.at[:, pl.ds(ub, 128)], mlpu_s[q], s3),
                    pltpu.async_copy(
                        mlpi_h.at[:, pl.ds(ib, 128)], mlpi_s[q], s4),
                )

            def body(jb, c2):
                g0 = h0 + jb * group
                a0 = jb * group
                uvec = uidx[pl.ds(g0, group)]
                ivec = iidx[pl.ds(g0, group)]
                ublk = (uvec >> 7) << 7
                iblk = (ivec >> 7) << 7
                ulane = uvec & 127
                ilane = ivec & 127
                cps = fire(ublk, iblk, 0)
                cps_next = fire(ublk, iblk, 1)
                for k in range(group):
                    j = a0 + k
                    q = k % 3
                    c1, cc2, c3, c4 = cps
                    cps = cps_next
                    if k + 2 < group:
                        cps_next = fire(ublk, iblk, k + 2)
                    ul = jnp.full((16,), ulane[k], jnp.int32)
                    il = jnp.full((16,), ilane[k], jnp.int32)
                    c1.wait()
                    vmfu = plsc.load_gather(mfu_s[q], [rows8, ul])
                    cc2.wait()
                    vmfi = plsc.load_gather(mfi_s[q], [rows8, il])
                    amf[j, pl.ds(0, 16)] = jnp.where(rows16 < 8, vmfu, vmfi)
                    c3.wait()
                    amlpu[j, pl.ds(0, 16)] = plsc.load_gather(
                        mlpu_s[q], [rows16, ul])
                    amlpu[j, pl.ds(16, 16)] = plsc.load_gather(
                        mlpu_s[q], [rows16 + 16, ul])
                    c4.wait()
                    amlpi[j, pl.ds(0, 16)] = plsc.load_gather(
                        mlpi_s[q], [rows16, il])
                    amlpi[j, pl.ds(16, 16)] = plsc.load_gather(
                        mlpi_s[q], [rows16 + 16, il])
                return c2

            lax.fori_loop(0, half // group, body, 0)
            pltpu.sync_copy(amf, mf_o.at[pl.ds(base + h0, half)])
            pltpu.sync_copy(amlpu, mlpu_o.at[pl.ds(base + h0, half)])
            pltpu.sync_copy(amlpi, mlpi_o.at[pl.ds(base + h0, half)])
            return carry

        lax.fori_loop(0, 2, half_pass, 0)

    return gather_kernel


def _mlp_body(mf_ui, mlpu, mlpi, W1, b1, W2, b2, W3, b3, W4, b4, Wp, bp,
              out, *, d_mf, d_mlp):
    h = (mlpu[...] @ W1[0:d_mlp, :] + mlpi[...] @ W1[d_mlp:2 * d_mlp, :]
         + b1[...])
    h = jnp.maximum(h, 0.0)
    h = jnp.maximum(h @ W2[...] + b2[...], 0.0)
    h = jnp.maximum(h @ W3[...] + b3[...], 0.0)
    h = jnp.maximum(h @ W4[...] + b4[...], 0.0)
    mf = mf_ui[:, 0:d_mf] * mf_ui[:, d_mf:2 * d_mf]
    logit = mf @ Wp[0:d_mf, :] + h @ Wp[d_mf:, :] + bp[...]
    out[...] = 1.0 / (1.0 + jnp.exp(-logit))


def kernel(user, item, additional_features, mf_user_emb, mf_item_emb,
           mlp_user_emb, mlp_item_emb, W1, b1, W2, b2, W3, b3, W4, b4,
           Wp, bp):
    del additional_features
    B = user.shape[0]
    d_mf = mf_user_emb.shape[1]
    d_mlp = mlp_user_emb.shape[1]

    gather = _make_sc_gather(B, d_mf, d_mlp, group=16)
    mf_ui, mlpu, mlpi = gather(
        user, item, mf_user_emb.T, mf_item_emb.T,
        mlp_user_emb.T, mlp_item_emb.T)

    blk = 2048
    full = lambda shape: pl.BlockSpec(shape, lambda i: (0, 0))
    body = functools.partial(_mlp_body, d_mf=d_mf, d_mlp=d_mlp)
    out = pl.pallas_call(
        body,
        grid=(B // blk,),
        in_specs=[
            pl.BlockSpec((blk, 16), lambda i: (i, 0)),
            pl.BlockSpec((blk, d_mlp), lambda i: (i, 0)),
            pl.BlockSpec((blk, d_mlp), lambda i: (i, 0)),
            full(W1.shape), full((1, b1.shape[0])),
            full(W2.shape), full((1, b2.shape[0])),
            full(W3.shape), full((1, b3.shape[0])),
            full(W4.shape), full((1, b4.shape[0])),
            full(Wp.shape), full((1, 1)),
        ],
        out_specs=pl.BlockSpec((blk, 1), lambda i: (i, 0)),
        out_shape=jax.ShapeDtypeStruct((B, 1), jnp.float32),
    )(mf_ui, mlpu, mlpi,
      W1, b1.reshape(1, -1), W2, b2.reshape(1, -1),
      W3, b3.reshape(1, -1), W4, b4.reshape(1, -1),
      Wp, bp.reshape(1, 1))
    return out.reshape(-1)


# speedup vs baseline: 5.9876x; 1.0790x over previous
import functools

import jax
import jax.numpy as jnp
from jax import lax
from jax.experimental import pallas as pl
from jax.experimental.pallas import tpu as pltpu
from jax.experimental.pallas import tpu_sc as plsc


def _make_sc_gather(B, d_mf, d_mlp, group):
    info = plsc.get_sparse_core_info()
    nw = info.num_cores * info.num_subcores
    b_per_w = B // nw
    half = b_per_w // 4
    mesh = plsc.VectorSubcoreMesh(core_axis_name="c", subcore_axis_name="s")

    f32 = jnp.float32
    out_type = [
        jax.ShapeDtypeStruct((B, 16), f32),
        jax.ShapeDtypeStruct((B, d_mlp), f32),
        jax.ShapeDtypeStruct((B, d_mlp), f32),
    ]

    @functools.partial(
        pl.kernel,
        out_type=out_type,
        mesh=mesh,
        compiler_params=pltpu.CompilerParams(use_tc_tiling_on_sc=True,
                                             needs_layout_passes=False),
        scratch_types=[
            pltpu.VMEM((b_per_w,), jnp.int32),
            pltpu.VMEM((b_per_w,), jnp.int32),
            [pltpu.VMEM((d_mf, 128), f32)] * 4,
            [pltpu.VMEM((d_mf, 128), f32)] * 4,
            [pltpu.VMEM((d_mlp, 128), f32)] * 4,
            [pltpu.VMEM((d_mlp, 128), f32)] * 4,
            pltpu.VMEM((half, 16), f32),
            pltpu.VMEM((half, d_mlp), f32),
            pltpu.VMEM((half, d_mlp), f32),
            pltpu.SemaphoreType.DMA,
            pltpu.SemaphoreType.DMA,
            pltpu.SemaphoreType.DMA,
            pltpu.SemaphoreType.DMA,
        ],
    )
    def gather_kernel(user_h, item_h, mfu_h, mfi_h, mlpu_h, mlpi_h,
                      mf_o, mlpu_o, mlpi_o,
                      uidx, iidx, mfu_s, mfi_s, mlpu_s, mlpi_s,
                      amf, amlpu, amlpi,
                      s1, s2, s3, s4):
        wid = lax.axis_index("s") * info.num_cores + lax.axis_index("c")
        base = wid * b_per_w
        pltpu.sync_copy(user_h.at[pl.ds(base, b_per_w)], uidx)
        pltpu.sync_copy(item_h.at[pl.ds(base, b_per_w)], iidx)

        rows16 = lax.iota(jnp.int32, 16)
        rows8 = rows16 & 7

        def half_pass(h, carry):
            h0 = h * half

            def fire(ublk, iblk, k):
                ub = pl.multiple_of(ublk[k], 128)
                ib = pl.multiple_of(iblk[k], 128)
                q = k % 4
                return (
                    pltpu.async_copy(
                        mfu_h.at[:, pl.ds(ub, 128)], mfu_s[q], s1),
                    pltpu.async_copy(
                        mfi_h.at[:, pl.ds(ib, 128)], mfi_s[q], s2),
                    pltpu.async_copy(
                        mlpu_h.at[:, pl.ds(ub, 128)], mlpu_s[q], s3),
                    pltpu.async_copy(
                        mlpi_h.at[:, pl.ds(ib, 128)], mlpi_s[q], s4),
                )

            def body(jb, c2):
                g0 = h0 + jb * group
                a0 = jb * group
                uvec = uidx[pl.ds(g0, group)]
                ivec = iidx[pl.ds(g0, group)]
                ublk = (uvec >> 7) << 7
                iblk = (ivec >> 7) << 7
                ulane = uvec & 127
                ilane = ivec & 127
                cps = fire(ublk, iblk, 0)
                cps1 = fire(ublk, iblk, 1)
                cps2 = fire(ublk, iblk, 2)
                for k in range(group):
                    j = a0 + k
                    q = k % 4
                    c1, cc2, c3, c4 = cps
                    cps = cps1
                    cps1 = cps2
                    if k + 3 < group:
                        cps2 = fire(ublk, iblk, k + 3)
                    ul = jnp.full((16,), ulane[k], jnp.int32)
                    il = jnp.full((16,), ilane[k], jnp.int32)
                    c1.wait()
                    vmfu = plsc.load_gather(mfu_s[q], [rows8, ul])
                    cc2.wait()
                    vmfi = plsc.load_gather(mfi_s[q], [rows8, il])
                    amf[j, pl.ds(0, 16)] = jnp.where(rows16 < 8, vmfu, vmfi)
                    c3.wait()
                    amlpu[j, pl.ds(0, 16)] = plsc.load_gather(
                        mlpu_s[q], [rows16, ul])
                    amlpu[j, pl.ds(16, 16)] = plsc.load_gather(
                        mlpu_s[q], [rows16 + 16, ul])
                    c4.wait()
                    amlpi[j, pl.ds(0, 16)] = plsc.load_gather(
                        mlpi_s[q], [rows16, il])
                    amlpi[j, pl.ds(16, 16)] = plsc.load_gather(
                        mlpi_s[q], [rows16 + 16, il])
                return c2

            lax.fori_loop(0, half // group, body, 0)
            pltpu.sync_copy(amf, mf_o.at[pl.ds(base + h0, half)])
            pltpu.sync_copy(amlpu, mlpu_o.at[pl.ds(base + h0, half)])
            pltpu.sync_copy(amlpi, mlpi_o.at[pl.ds(base + h0, half)])
            return carry

        lax.fori_loop(0, 4, half_pass, 0)

    return gather_kernel


def _mlp_body(mf_ui, mlpu, mlpi, W1, b1, W2, b2, W3, b3, W4, b4, Wp, bp,
              out, *, d_mf, d_mlp):
    h = (mlpu[...] @ W1[0:d_mlp, :] + mlpi[...] @ W1[d_mlp:2 * d_mlp, :]
         + b1[...])
    h = jnp.maximum(h, 0.0)
    h = jnp.maximum(h @ W2[...] + b2[...], 0.0)
    h = jnp.maximum(h @ W3[...] + b3[...], 0.0)
    h = jnp.maximum(h @ W4[...] + b4[...], 0.0)
    mf = mf_ui[:, 0:d_mf] * mf_ui[:, d_mf:2 * d_mf]
    logit = mf @ Wp[0:d_mf, :] + h @ Wp[d_mf:, :] + bp[...]
    out[...] = 1.0 / (1.0 + jnp.exp(-logit))


def kernel(user, item, additional_features, mf_user_emb, mf_item_emb,
           mlp_user_emb, mlp_item_emb, W1, b1, W2, b2, W3, b3, W4, b4,
           Wp, bp):
    del additional_features
    B = user.shape[0]
    d_mf = mf_user_emb.shape[1]
    d_mlp = mlp_user_emb.shape[1]

    gather = _make_sc_gather(B, d_mf, d_mlp, group=16)
    mf_ui, mlpu, mlpi = gather(
        user, item, mf_user_emb.T, mf_item_emb.T,
        mlp_user_emb.T, mlp_item_emb.T)

    blk = 2048
    full = lambda shape: pl.BlockSpec(shape, lambda i: (0, 0))
    body = functools.partial(_mlp_body, d_mf=d_mf, d_mlp=d_mlp)
    out = pl.pallas_call(
        body,
        grid=(B // blk,),
        in_specs=[
            pl.BlockSpec((blk, 16), lambda i: (i, 0)),
            pl.BlockSpec((blk, d_mlp), lambda i: (i, 0)),
            pl.BlockSpec((blk, d_mlp), lambda i: (i, 0)),
            full(W1.shape), full((1, b1.shape[0])),
            full(W2.shape), full((1, b2.shape[0])),
            full(W3.shape), full((1, b3.shape[0])),
            full(W4.shape), full((1, b4.shape[0])),
            full(Wp.shape), full((1, 1)),
        ],
        out_specs=pl.BlockSpec((blk, 1), lambda i: (i, 0)),
        out_shape=jax.ShapeDtypeStruct((B, 1), jnp.float32),
    )(mf_ui, mlpu, mlpi,
      W1, b1.reshape(1, -1), W2, b2.reshape(1, -1),
      W3, b3.reshape(1, -1), W4, b4.reshape(1, -1),
      Wp, bp.reshape(1, 1))
    return out.reshape(-1)


# speedup vs baseline: 6.3105x; 1.0539x over previous
import functools

import jax
import jax.numpy as jnp
from jax import lax
from jax.experimental import pallas as pl
from jax.experimental.pallas import tpu as pltpu
from jax.experimental.pallas import tpu_sc as plsc


def _make_sc_gather(B, d_mf, d_mlp, group):
    info = plsc.get_sparse_core_info()
    nw = info.num_cores * info.num_subcores
    b_per_w = B // nw
    half = b_per_w // 8
    mesh = plsc.VectorSubcoreMesh(core_axis_name="c", subcore_axis_name="s")

    f32 = jnp.float32
    out_type = [
        jax.ShapeDtypeStruct((B, 16), f32),
        jax.ShapeDtypeStruct((B, d_mlp), f32),
        jax.ShapeDtypeStruct((B, d_mlp), f32),
    ]

    @functools.partial(
        pl.kernel,
        out_type=out_type,
        mesh=mesh,
        compiler_params=pltpu.CompilerParams(use_tc_tiling_on_sc=True,
                                             needs_layout_passes=False),
        scratch_types=[
            pltpu.VMEM((b_per_w,), jnp.int32),
            pltpu.VMEM((b_per_w,), jnp.int32),
            [pltpu.VMEM((d_mf, 128), f32)] * 5,
            [pltpu.VMEM((d_mf, 128), f32)] * 5,
            [pltpu.VMEM((d_mlp, 128), f32)] * 5,
            [pltpu.VMEM((d_mlp, 128), f32)] * 5,
            pltpu.VMEM((half, 16), f32),
            pltpu.VMEM((half, d_mlp), f32),
            pltpu.VMEM((half, d_mlp), f32),
            pltpu.SemaphoreType.DMA,
            pltpu.SemaphoreType.DMA,
            pltpu.SemaphoreType.DMA,
            pltpu.SemaphoreType.DMA,
        ],
    )
    def gather_kernel(user_h, item_h, mfu_h, mfi_h, mlpu_h, mlpi_h,
                      mf_o, mlpu_o, mlpi_o,
                      uidx, iidx, mfu_s, mfi_s, mlpu_s, mlpi_s,
                      amf, amlpu, amlpi,
                      s1, s2, s3, s4):
        wid = lax.axis_index("s") * info.num_cores + lax.axis_index("c")
        base = wid * b_per_w
        pltpu.sync_copy(user_h.at[pl.ds(base, b_per_w)], uidx)
        pltpu.sync_copy(item_h.at[pl.ds(base, b_per_w)], iidx)

        rows16 = lax.iota(jnp.int32, 16)
        rows8 = rows16 & 7

        def half_pass(h, carry):
            h0 = h * half

            def fire(ublk, iblk, k):
                ub = pl.multiple_of(ublk[k], 128)
                ib = pl.multiple_of(iblk[k], 128)
                q = k % 5
                return (
                    pltpu.async_copy(
                        mfu_h.at[:, pl.ds(ub, 128)], mfu_s[q], s1),
                    pltpu.async_copy(
                        mfi_h.at[:, pl.ds(ib, 128)], mfi_s[q], s2),
                    pltpu.async_copy(
                        mlpu_h.at[:, pl.ds(ub, 128)], mlpu_s[q], s3),
                    pltpu.async_copy(
                        mlpi_h.at[:, pl.ds(ib, 128)], mlpi_s[q], s4),
                )

            def body(jb, c2):
                g0 = h0 + jb * group
                a0 = jb * group
                uvec = uidx[pl.ds(g0, group)]
                ivec = iidx[pl.ds(g0, group)]
                ublk = (uvec >> 7) << 7
                iblk = (ivec >> 7) << 7
                ulane = uvec & 127
                ilane = ivec & 127
                cps = fire(ublk, iblk, 0)
                cps1 = fire(ublk, iblk, 1)
                cps2 = fire(ublk, iblk, 2)
                cps3 = fire(ublk, iblk, 3)
                for k in range(group):
                    j = a0 + k
                    q = k % 5
                    c1, cc2, c3, c4 = cps
                    cps = cps1
                    cps1 = cps2
                    cps2 = cps3
                    if k + 4 < group:
                        cps3 = fire(ublk, iblk, k + 4)
                    ul = jnp.full((16,), ulane[k], jnp.int32)
                    il = jnp.full((16,), ilane[k], jnp.int32)
                    c1.wait()
                    vmfu = plsc.load_gather(mfu_s[q], [rows8, ul])
                    cc2.wait()
                    vmfi = plsc.load_gather(mfi_s[q], [rows8, il])
                    amf[j, pl.ds(0, 16)] = jnp.where(rows16 < 8, vmfu, vmfi)
                    c3.wait()
                    amlpu[j, pl.ds(0, 16)] = plsc.load_gather(
                        mlpu_s[q], [rows16, ul])
                    amlpu[j, pl.ds(16, 16)] = plsc.load_gather(
                        mlpu_s[q], [rows16 + 16, ul])
                    c4.wait()
                    amlpi[j, pl.ds(0, 16)] = plsc.load_gather(
                        mlpi_s[q], [rows16, il])
                    amlpi[j, pl.ds(16, 16)] = plsc.load_gather(
                        mlpi_s[q], [rows16 + 16, il])
                return c2

            lax.fori_loop(0, half // group, body, 0)
            pltpu.sync_copy(amf, mf_o.at[pl.ds(base + h0, half)])
            pltpu.sync_copy(amlpu, mlpu_o.at[pl.ds(base + h0, half)])
            pltpu.sync_copy(amlpi, mlpi_o.at[pl.ds(base + h0, half)])
            return carry

        lax.fori_loop(0, 8, half_pass, 0)

    return gather_kernel


def _mlp_body(mf_ui, mlpu, mlpi, W1, b1, W2, b2, W3, b3, W4, b4, Wp, bp,
              out, *, d_mf, d_mlp):
    h = (mlpu[...] @ W1[0:d_mlp, :] + mlpi[...] @ W1[d_mlp:2 * d_mlp, :]
         + b1[...])
    h = jnp.maximum(h, 0.0)
    h = jnp.maximum(h @ W2[...] + b2[...], 0.0)
    h = jnp.maximum(h @ W3[...] + b3[...], 0.0)
    h = jnp.maximum(h @ W4[...] + b4[...], 0.0)
    mf = mf_ui[:, 0:d_mf] * mf_ui[:, d_mf:2 * d_mf]
    logit = mf @ Wp[0:d_mf, :] + h @ Wp[d_mf:, :] + bp[...]
    out[...] = 1.0 / (1.0 + jnp.exp(-logit))


def kernel(user, item, additional_features, mf_user_emb, mf_item_emb,
           mlp_user_emb, mlp_item_emb, W1, b1, W2, b2, W3, b3, W4, b4,
           Wp, bp):
    del additional_features
    B = user.shape[0]
    d_mf = mf_user_emb.shape[1]
    d_mlp = mlp_user_emb.shape[1]

    gather = _make_sc_gather(B, d_mf, d_mlp, group=16)
    mf_ui, mlpu, mlpi = gather(
        user, item, mf_user_emb.T, mf_item_emb.T,
        mlp_user_emb.T, mlp_item_emb.T)

    blk = 2048
    full = lambda shape: pl.BlockSpec(shape, lambda i: (0, 0))
    body = functools.partial(_mlp_body, d_mf=d_mf, d_mlp=d_mlp)
    out = pl.pallas_call(
        body,
        grid=(B // blk,),
        in_specs=[
            pl.BlockSpec((blk, 16), lambda i: (i, 0)),
            pl.BlockSpec((blk, d_mlp), lambda i: (i, 0)),
            pl.BlockSpec((blk, d_mlp), lambda i: (i, 0)),
            full(W1.shape), full((1, b1.shape[0])),
            full(W2.shape), full((1, b2.shape[0])),
            full(W3.shape), full((1, b3.shape[0])),
            full(W4.shape), full((1, b4.shape[0])),
            full(Wp.shape), full((1, 1)),
        ],
        out_specs=pl.BlockSpec((blk, 1), lambda i: (i, 0)),
        out_shape=jax.ShapeDtypeStruct((B, 1), jnp.float32),
    )(mf_ui, mlpu, mlpi,
      W1, b1.reshape(1, -1), W2, b2.reshape(1, -1),
      W3, b3.reshape(1, -1), W4, b4.reshape(1, -1),
      Wp, bp.reshape(1, 1))
    return out.reshape(-1)
